# Initial kernel scaffold; baseline (speedup 1.0000x reference)
import jax
import jax.numpy as jnp
from jax.experimental import pallas as pl


def kernel(words, emits):
    raise NotImplementedError("write your pallas kernel here")


# speedup vs baseline: 2.1458x; 2.1458x over previous
import functools

import jax
import jax.numpy as jnp
from jax import lax
from jax.experimental import pallas as pl
from jax.experimental.pallas import tpu as pltpu
from jax.experimental.pallas import tpu_sc as plsc

N_TAGS = 48
N_WORDS = 100000
B = 1024
L = 200
N_TOK = B * L

_TRB = 512
_TR_GRID = (N_WORDS + _TRB - 1) // _TRB


def _tr_body(x_ref, o_ref):
    o_ref[...] = x_ref[...].T


_transpose = pl.pallas_call(
    _tr_body,
    grid=(_TR_GRID,),
    in_specs=[pl.BlockSpec((N_TAGS, _TRB), lambda i: (0, i))],
    out_specs=pl.BlockSpec((_TRB, N_TAGS), lambda i: (i, 0)),
    out_shape=jax.ShapeDtypeStruct((N_WORDS, N_TAGS), jnp.float32),
)

_NC = 2
_NS = 16
_NW = _NC * _NS
_CHUNK = 128
_NCHUNKS = N_TOK // _CHUNK
_CPW = _NCHUNKS // _NW

_sc_mesh = plsc.VectorSubcoreMesh(core_axis_name="c", subcore_axis_name="s")


@functools.partial(
    pl.kernel,
    out_type=jax.ShapeDtypeStruct((N_TOK, N_TAGS), jnp.float32),
    mesh=_sc_mesh,
    scratch_types=[
        pltpu.VMEM((_CPW, _CHUNK), jnp.int32),
        pltpu.VMEM((_CHUNK, N_TAGS), jnp.float32),
        pltpu.SemaphoreType.DMA,
    ],
    compiler_params=pltpu.CompilerParams(use_tc_tiling_on_sc=False),
)
def _sc_gather(table_hbm, idx_hbm, out_hbm, idx_v, rows_v, sem):
    wid = lax.axis_index("s") * _NC + lax.axis_index("c")
    base = wid * _CPW
    pltpu.sync_copy(idx_hbm.at[wid], idx_v)

    def step(j, carry):
        pltpu.async_copy(table_hbm.at[idx_v.at[j]], rows_v, sem).wait()
        pltpu.sync_copy(rows_v, out_hbm.at[pl.ds((base + j) * _CHUNK, _CHUNK)])
        return carry

    lax.fori_loop(0, _CPW, step, 0)


def kernel(words, emits):
    table = _transpose(emits)
    idx = words.reshape(_NW, _CPW, _CHUNK)
    out = _sc_gather(table, idx)
    return out.reshape(B, L, N_TAGS)


# speedup vs baseline: 2.3822x; 1.1102x over previous
import functools

import jax
import jax.numpy as jnp
from jax import lax
from jax.experimental import pallas as pl
from jax.experimental.pallas import tpu as pltpu
from jax.experimental.pallas import tpu_sc as plsc

N_TAGS = 48
N_WORDS = 100000
B = 1024
L = 200

_TRB = 512
_TR_GRID = (N_WORDS + _TRB - 1) // _TRB


def _tr_body(x_ref, o_ref):
    o_ref[...] = x_ref[...].T


_transpose = pl.pallas_call(
    _tr_body,
    grid=(_TR_GRID,),
    in_specs=[pl.BlockSpec((N_TAGS, _TRB), lambda i: (0, i))],
    out_specs=pl.BlockSpec((_TRB, N_TAGS), lambda i: (i, 0)),
    out_shape=jax.ShapeDtypeStruct((N_WORDS, N_TAGS), jnp.float32),
)

_NC = 2
_NS = 16
_NW = _NC * _NS
_SPW = B // _NW
_GSPLIT = (104, 96)
_BS = 4
_NBLK = _SPW // _BS

_sc_mesh = plsc.VectorSubcoreMesh(core_axis_name="c", subcore_axis_name="s")


@functools.partial(
    pl.kernel,
    out_type=jax.ShapeDtypeStruct((B, L, N_TAGS), jnp.float32),
    mesh=_sc_mesh,
    scratch_types=[
        pltpu.VMEM((_SPW, L), jnp.int32),
        pltpu.VMEM((_BS, L, N_TAGS), jnp.float32),
        pltpu.VMEM((_BS, L, N_TAGS), jnp.float32),
        pltpu.VMEM((_BS, L, N_TAGS), jnp.float32),
        pltpu.SemaphoreType.DMA,
        pltpu.SemaphoreType.DMA,
    ],
    compiler_params=pltpu.CompilerParams(use_tc_tiling_on_sc=False),
)
def _sc_gather(table_hbm, words_hbm, out_hbm, idx_v, b0, b1, b2, sem_g, sem_s):
    wid = lax.axis_index("s") * _NC + lax.axis_index("c")
    sent0 = wid * _SPW
    bufs = (b0, b1, b2)
    pltpu.sync_copy(words_hbm.at[pl.ds(sent0, _SPW)], idx_v)

    def fire_block(g):
        buf = bufs[g % 3]
        cps = []
        for s in range(_BS):
            off = 0
            for sz in _GSPLIT:
                idx = idx_v.at[g * _BS + s, pl.ds(off, sz)]
                dst = buf.at[s, pl.ds(off, sz)]
                cps.append(pltpu.async_copy(table_hbm.at[idx], dst, sem_g))
                off += sz
        return cps

    gathers = [None] * _NBLK
    stores = [None] * _NBLK
    gathers[0] = fire_block(0)
    gathers[1] = fire_block(1)
    for g in range(_NBLK):
        for cp in gathers[g]:
            cp.wait()
        stores[g] = pltpu.async_copy(
            bufs[g % 3], out_hbm.at[pl.ds(sent0 + g * _BS, _BS)], sem_s)
        if g + 2 < _NBLK:
            if g >= 1:
                stores[g - 1].wait()
            gathers[g + 2] = fire_block(g + 2)
    stores[_NBLK - 2].wait()
    stores[_NBLK - 1].wait()


def kernel(words, emits):
    table = _transpose(emits)
    return _sc_gather(table, words)


# speedup vs baseline: 3.3201x; 1.3937x over previous
import functools

import jax
import jax.numpy as jnp
from jax import lax
from jax.experimental import pallas as pl
from jax.experimental.pallas import tpu as pltpu
from jax.experimental.pallas import tpu_sc as plsc

N_TAGS = 48
N_WORDS = 100000
B = 1024
L = 200

_NC = 2
_NS = 16
_NW = _NC * _NS
_LANES = 16

_sc_mesh = plsc.VectorSubcoreMesh(core_axis_name="c", subcore_axis_name="s")
_sc_params = pltpu.CompilerParams(use_tc_tiling_on_sc=False)
_sc_params_nlp = pltpu.CompilerParams(
    use_tc_tiling_on_sc=False, needs_layout_passes=False)

_WPW = 3136
N_WORDS_PAD = _NW * _WPW
_WB = 448
_NBLK_T = _WPW // _WB
_WG = _WB // _LANES


@functools.partial(
    pl.kernel,
    out_type=jax.ShapeDtypeStruct((N_WORDS_PAD, N_TAGS), jnp.float32),
    mesh=_sc_mesh,
    scratch_types=[
        pltpu.VMEM((N_TAGS, _WB), jnp.float32),
        pltpu.VMEM((N_TAGS, _WB), jnp.float32),
        pltpu.VMEM((_WB, N_TAGS), jnp.float32),
        pltpu.VMEM((_WB, N_TAGS), jnp.float32),
        pltpu.SemaphoreType.DMA,
        pltpu.SemaphoreType.DMA,
    ],
    compiler_params=_sc_params_nlp,
)
def _sc_transpose(emits_hbm, table_hbm, i0, i1, o0, o1, sem_in, sem_out):
    wid = lax.axis_index("s") * _NC + lax.axis_index("c")
    ivs = (i0, i1)
    ovs = (o0, o1)
    iota = lax.iota(jnp.int32, _LANES)

    def _w0(b):
        return jnp.minimum(wid * _WPW + b * _WB, N_WORDS - _WB)

    def fire_in(b):
        iv = ivs[b % 2]
        w0 = _w0(b)

        def body(t, carry):
            pltpu.async_copy(emits_hbm.at[t, pl.ds(w0, _WB)], iv.at[t], sem_in)
            return carry

        lax.fori_loop(0, N_TAGS, body, 0)

    def wait_in(b):
        pltpu.make_async_copy(
            emits_hbm.at[0, pl.ds(0, _WB)], ivs[b % 2].at[0], sem_in
        ).wait()

    def transpose(b):
        iv, ov = ivs[b % 2], ovs[b % 2]

        def body(wg, carry):
            rows = wg * _LANES + iota
            for t in range(N_TAGS):
                v = iv[t, pl.ds(wg * _LANES, _LANES)]
                plsc.store_scatter(ov, [rows, jnp.full((_LANES,), t, jnp.int32)], v)
            return carry

        lax.fori_loop(0, _WG, body, 0)

    stores = [None] * _NBLK_T
    fire_in(0)
    for b in range(_NBLK_T):
        for _ in range(N_TAGS):
            wait_in(b)
        if b + 1 < _NBLK_T:
            fire_in(b + 1)
        if b >= 2:
            stores[b - 2].wait()
        transpose(b)
        stores[b] = pltpu.async_copy(
            ovs[b % 2], table_hbm.at[pl.ds(_w0(b), _WB)], sem_out)
    stores[_NBLK_T - 2].wait()
    stores[_NBLK_T - 1].wait()


_SPW = B // _NW
_GSPLIT = (104, 96)
_BS = 4
_NBLK = _SPW // _BS


@functools.partial(
    pl.kernel,
    out_type=jax.ShapeDtypeStruct((B, L, N_TAGS), jnp.float32),
    mesh=_sc_mesh,
    scratch_types=[
        pltpu.VMEM((_SPW, L), jnp.int32),
        pltpu.VMEM((_BS, L, N_TAGS), jnp.float32),
        pltpu.VMEM((_BS, L, N_TAGS), jnp.float32),
        pltpu.VMEM((_BS, L, N_TAGS), jnp.float32),
        pltpu.SemaphoreType.DMA,
        pltpu.SemaphoreType.DMA,
    ],
    compiler_params=_sc_params,
)
def _sc_gather(table_hbm, words_hbm, out_hbm, idx_v, b0, b1, b2, sem_g, sem_s):
    wid = lax.axis_index("s") * _NC + lax.axis_index("c")
    sent0 = wid * _SPW
    bufs = (b0, b1, b2)
    pltpu.sync_copy(words_hbm.at[pl.ds(sent0, _SPW)], idx_v)

    def fire_block(g):
        buf = bufs[g % 3]
        cps = []
        for s in range(_BS):
            off = 0
            for sz in _GSPLIT:
                idx = idx_v.at[g * _BS + s, pl.ds(off, sz)]
                dst = buf.at[s, pl.ds(off, sz)]
                cps.append(pltpu.async_copy(table_hbm.at[idx], dst, sem_g))
                off += sz
        return cps

    gathers = [None] * _NBLK
    stores = [None] * _NBLK
    gathers[0] = fire_block(0)
    gathers[1] = fire_block(1)
    for g in range(_NBLK):
        for cp in gathers[g]:
            cp.wait()
        stores[g] = pltpu.async_copy(
            bufs[g % 3], out_hbm.at[pl.ds(sent0 + g * _BS, _BS)], sem_s)
        if g + 2 < _NBLK:
            if g >= 1:
                stores[g - 1].wait()
            gathers[g + 2] = fire_block(g + 2)
    stores[_NBLK - 2].wait()
    stores[_NBLK - 1].wait()


def kernel(words, emits):
    table = _sc_transpose(emits)
    return _sc_gather(table, words)
